# baseline (speedup 1.0000x reference)
import functools

import jax
import jax.numpy as jnp
from jax import lax
from jax.experimental import pallas as pl
from jax.experimental.pallas import tpu as pltpu
from jax.experimental.pallas import tpu_sc as plsc

VOCAB = 1000000
EMBED_DIM = 32
BATCH = 16384
FIELDS = 26
B_TOTAL = BATCH * FIELDS

NC = 2
NS = 16
NW = NC * NS

LANES = 128
PACK = LANES // EMBED_DIM
RT_TILES = -(-VOCAB // LANES)
CT_TILES = EMBED_DIM // 8
DST_ROWS = 32 * RT_TILES
VOCAB_PAD = DST_ROWS * PACK
RT_PER_W = -(-RT_TILES // NW)

BT_TILES = BATCH // LANES
UNITS = FIELDS * BT_TILES
U_PER_W = UNITS // NW

_MESH = dict(core_axis_name="c", subcore_axis_name="s")


def _diag_retile():
    iota = lax.iota(jnp.int32, 16)
    out = []
    for a in (0, 16):
        ic = (a + iota) & 31
        for r0 in range(128):
            irin = (r0 + iota) & 127
            out.append((ic, irin, irin >> 2, ((irin & 3) << 5) + ic))
    return out


@functools.partial(
    pl.kernel,
    mesh=plsc.VectorSubcoreMesh(**_MESH),
    out_type=jax.ShapeDtypeStruct((DST_ROWS, LANES), jnp.float32),
    scratch_types=[
        pltpu.VMEM((2, EMBED_DIM, LANES), jnp.float32),
        pltpu.VMEM((2, 32, LANES), jnp.float32),
        pltpu.SemaphoreType.DMA,
        pltpu.SemaphoreType.DMA,
    ],
    compiler_params=pltpu.CompilerParams(needs_layout_passes=False),
)
def _retile_sc(t_hbm, out_hbm, src_v, dst_v, sem_in, sem_out):
    wid = lax.axis_index("s") * NC + lax.axis_index("c")
    base = wid * RT_PER_W
    diag = _diag_retile()

    TAIL = VOCAB - (RT_TILES - 1) * LANES

    def in_descs(g, b, width):
        rt = base + g
        return [
            pltpu.make_async_copy(
                t_hbm.at[pl.ds(8 * ct, 8), pl.ds(rt * LANES, width)],
                src_v.at[b, pl.ds(8 * ct, 8), pl.ds(0, width)],
                sem_in,
            )
            for ct in range(CT_TILES)
        ]

    def out_desc(g, b):
        rt = base + g
        return pltpu.make_async_copy(
            dst_v.at[b], out_hbm.at[pl.ds(rt * 32, 32)], sem_out
        )

    def valid(g):
        return (g < RT_PER_W) & (base + g < RT_TILES)

    def in_all(g, b, fn):
        @pl.when(valid(g) & (base + g < RT_TILES - 1))
        def _full():
            for d in in_descs(g, b, LANES):
                fn(d)

        @pl.when(base + g == RT_TILES - 1)
        def _tail():
            for d in in_descs(g, b, TAIL):
                fn(d)

    in_all(jnp.int32(0), 0, lambda d: d.start())

    def process(g, b):
        in_all(g, b, lambda d: d.wait())
        in_all(g + 1, 1 - b, lambda d: d.start())

        @pl.when((g >= 2) & valid(g - 2))
        def _wait_out():
            out_desc(g - 2, b).wait()

        @pl.when(valid(g))
        def _repack():
            for ic, irin, ip, il in diag:
                vals = plsc.load_gather(src_v.at[b], [ic, irin])
                plsc.store_scatter(dst_v.at[b], [ip, il], vals)
            out_desc(g, b).start()

    def body(i, carry):
        process(2 * i, 0)
        process(2 * i + 1, 1)
        return carry

    lax.fori_loop(0, (RT_PER_W + 1) // 2 + 1, body, 0)


def _diag_gather():
    iota = lax.iota(jnp.int32, 16)
    out = []
    for a in (0, 16):
        ic = (a + iota) & 31
        for b0 in range(128):
            ibin = (b0 + iota) & 127
            out.append((ibin, ic))
    return out


@functools.partial(
    pl.kernel,
    mesh=plsc.VectorSubcoreMesh(**_MESH),
    out_type=jax.ShapeDtypeStruct(
        (FIELDS, CT_TILES, BT_TILES, 8, LANES), jnp.float32
    ),
    scratch_types=[
        pltpu.VMEM((2, LANES), jnp.int32),
        pltpu.VMEM((2, LANES, EMBED_DIM), jnp.float32),
        pltpu.VMEM((2, EMBED_DIM, LANES), jnp.float32),
        pltpu.SemaphoreType.DMA,
        pltpu.SemaphoreType.DMA,
        pltpu.SemaphoreType.DMA,
    ],
    compiler_params=pltpu.CompilerParams(
        use_tc_tiling_on_sc=False, needs_layout_passes=False
    ),
)
def _embed_gather(
    idx_hbm, table_hbm, out_hbm, idx_v, rows_v, tile_v, sem_i, sem_g, sem_o
):
    wid = lax.axis_index("s") * NC + lax.axis_index("c")
    ubase = wid * U_PER_W
    diag = _diag_gather()

    def idx_desc(j, b):
        u = ubase + j
        return pltpu.make_async_copy(
            idx_hbm.at[u >> 7, pl.ds((u & (BT_TILES - 1)) * LANES, LANES)],
            idx_v.at[b],
            sem_i,
        )

    def gat_desc(b):
        return pltpu.make_async_copy(
            table_hbm.at[idx_v.at[b]], rows_v.at[b], sem_g
        )

    def out_descs(j, b):
        u = ubase + j
        return [
            pltpu.make_async_copy(
                tile_v.at[b, pl.ds(8 * ct, 8)],
                out_hbm.at[u >> 7, ct, u & (BT_TILES - 1)],
                sem_o,
            )
            for ct in range(CT_TILES)
        ]

    idx_desc(jnp.int32(0), 0).start()
    idx_desc(jnp.int32(0), 0).wait()
    gat_desc(0).start()
    idx_desc(jnp.int32(1), 1).start()

    def process(j, b):
        @pl.when(j + 1 < U_PER_W)
        def _pref():
            idx_desc(j + 1, 1 - b).wait()
            gat_desc(1 - b).start()

        gat_desc(b).wait()

        @pl.when(j + 2 < U_PER_W)
        def _pref_idx():
            idx_desc(j + 2, b).start()

        @pl.when(j >= 2)
        def _wait_out():
            for d in out_descs(j - 2, b):
                d.wait()

        for ibin, ic in diag:
            vals = plsc.load_gather(rows_v.at[b], [ibin, ic])
            plsc.store_scatter(tile_v.at[b], [ic, ibin], vals)
        for d in out_descs(j, b):
            d.start()

    def body(i, carry):
        process(2 * i, 0)
        process(2 * i + 1, 1)
        return carry

    lax.fori_loop(0, U_PER_W // 2, body, 0)
    for j, b in ((U_PER_W - 2, 0), (U_PER_W - 1, 1)):
        for d in out_descs(jnp.int32(j), b):
            d.wait()


def kernel(indices, table):
    packed = _retile_sc(table.T)
    table_rm = packed.reshape(VOCAB_PAD, EMBED_DIM)
    idx_t = indices.T.astype(jnp.int32)
    out5 = _embed_gather(idx_t, table_rm)
    return out5.transpose((2, 4, 0, 1, 3)).reshape(BATCH, FIELDS, EMBED_DIM)


# speedup vs baseline: 1.2466x; 1.2466x over previous
import functools

import jax
import jax.numpy as jnp
from jax import lax
from jax.experimental import pallas as pl
from jax.experimental.pallas import tpu as pltpu
from jax.experimental.pallas import tpu_sc as plsc

VOCAB = 1000000
EMBED_DIM = 32
BATCH = 16384
FIELDS = 26
B_TOTAL = BATCH * FIELDS

NC = 2
NS = 16
NW = NC * NS

LANES = 128
PACK = LANES // EMBED_DIM
RT_TILES = -(-VOCAB // LANES)
CT_TILES = EMBED_DIM // 8
DST_ROWS = 32 * RT_TILES
VOCAB_PAD = DST_ROWS * PACK
RT_PER_W = -(-RT_TILES // NW)

BT_TILES = BATCH // LANES
UNITS = FIELDS * BT_TILES
U_PER_W = UNITS // NW

_MESH = dict(core_axis_name="c", subcore_axis_name="s")


def _diag_retile():
    iota = lax.iota(jnp.int32, 16)
    out = []
    for a in (0, 16):
        for s in range(16):
            icl = (s + iota) & 15
            out.append((a, icl, ((iota & 3) << 5) + a + icl))
    return out


@functools.partial(
    pl.kernel,
    mesh=plsc.VectorSubcoreMesh(**_MESH),
    out_type=jax.ShapeDtypeStruct((DST_ROWS, LANES), jnp.float32),
    scratch_types=[
        pltpu.VMEM((2, EMBED_DIM, LANES), jnp.float32),
        pltpu.VMEM((2, 32, LANES), jnp.float32),
        pltpu.SemaphoreType.DMA,
        pltpu.SemaphoreType.DMA,
    ],
    compiler_params=pltpu.CompilerParams(needs_layout_passes=False),
)
def _retile_sc(t_hbm, out_hbm, src_v, dst_v, sem_in, sem_out):
    wid = lax.axis_index("s") * NC + lax.axis_index("c")
    base = wid * RT_PER_W
    diag = _diag_retile()

    TAIL = VOCAB - (RT_TILES - 1) * LANES

    def in_descs(g, b, width):
        rt = base + g
        return [
            pltpu.make_async_copy(
                t_hbm.at[pl.ds(8 * ct, 8), pl.ds(rt * LANES, width)],
                src_v.at[b, pl.ds(8 * ct, 8), pl.ds(0, width)],
                sem_in,
            )
            for ct in range(CT_TILES)
        ]

    def out_desc(g, b):
        rt = base + g
        return pltpu.make_async_copy(
            dst_v.at[b], out_hbm.at[pl.ds(rt * 32, 32)], sem_out
        )

    def valid(g):
        return (g < RT_PER_W) & (base + g < RT_TILES)

    def in_all(g, b, fn):
        @pl.when(valid(g) & (base + g < RT_TILES - 1))
        def _full():
            for d in in_descs(g, b, LANES):
                fn(d)

        @pl.when(base + g == RT_TILES - 1)
        def _tail():
            for d in in_descs(g, b, TAIL):
                fn(d)

    in_all(jnp.int32(0), 0, lambda d: d.start())

    def process(g, b):
        in_all(g, b, lambda d: d.wait())
        in_all(g + 1, 1 - b, lambda d: d.start())

        @pl.when((g >= 2) & valid(g - 2))
        def _wait_out():
            out_desc(g - 2, b).wait()

        @pl.when(valid(g))
        def _repack():
            iota = lax.iota(jnp.int32, 16)
            irow = iota >> 2
            for q in range(8):
                irin = 16 * q + iota
                dv = dst_v.at[b, pl.ds(4 * q, 4), :]
                for a, icl, il in diag:
                    sv = src_v.at[b, pl.ds(a, 16), :]
                    vals = plsc.load_gather(sv, [icl, irin])
                    plsc.store_scatter(dv, [irow, il], vals)
            out_desc(g, b).start()

    def body(i, carry):
        process(2 * i, 0)
        process(2 * i + 1, 1)
        return carry

    lax.fori_loop(0, (RT_PER_W + 1) // 2 + 1, body, 0)


def _diag_gather():
    iota = lax.iota(jnp.int32, 16)
    out = []
    for a in (0, 16):
        for s in range(16):
            icl = (s + iota) & 15
            out.append((a, icl, a + icl))
    return out


@functools.partial(
    pl.kernel,
    mesh=plsc.VectorSubcoreMesh(**_MESH),
    out_type=jax.ShapeDtypeStruct(
        (FIELDS, CT_TILES, BT_TILES, 8, LANES), jnp.float32
    ),
    scratch_types=[
        pltpu.VMEM((2, LANES), jnp.int32),
        pltpu.VMEM((2, LANES, EMBED_DIM), jnp.float32),
        pltpu.VMEM((2, EMBED_DIM, LANES), jnp.float32),
        pltpu.SemaphoreType.DMA,
        pltpu.SemaphoreType.DMA,
        pltpu.SemaphoreType.DMA,
    ],
    compiler_params=pltpu.CompilerParams(
        use_tc_tiling_on_sc=False, needs_layout_passes=False
    ),
)
def _embed_gather(
    idx_hbm, table_hbm, out_hbm, idx_v, rows_v, tile_v, sem_i, sem_g, sem_o
):
    wid = lax.axis_index("s") * NC + lax.axis_index("c")
    ubase = wid * U_PER_W
    diag = _diag_gather()

    def idx_desc(j, b):
        u = ubase + j
        return pltpu.make_async_copy(
            idx_hbm.at[u >> 7, pl.ds((u & (BT_TILES - 1)) * LANES, LANES)],
            idx_v.at[b],
            sem_i,
        )

    def gat_desc(b):
        return pltpu.make_async_copy(
            table_hbm.at[idx_v.at[b]], rows_v.at[b], sem_g
        )

    def out_descs(j, b):
        u = ubase + j
        return [
            pltpu.make_async_copy(
                tile_v.at[b, pl.ds(8 * ct, 8)],
                out_hbm.at[u >> 7, ct, u & (BT_TILES - 1)],
                sem_o,
            )
            for ct in range(CT_TILES)
        ]

    idx_desc(jnp.int32(0), 0).start()
    idx_desc(jnp.int32(0), 0).wait()
    gat_desc(0).start()
    idx_desc(jnp.int32(1), 1).start()

    def process(j, b):
        @pl.when(j + 1 < U_PER_W)
        def _pref():
            idx_desc(j + 1, 1 - b).wait()
            gat_desc(1 - b).start()

        gat_desc(b).wait()

        @pl.when(j + 2 < U_PER_W)
        def _pref_idx():
            idx_desc(j + 2, b).start()

        @pl.when(j >= 2)
        def _wait_out():
            for d in out_descs(j - 2, b):
                d.wait()

        iota = lax.iota(jnp.int32, 16)
        for q in range(8):
            rv = rows_v.at[b, pl.ds(16 * q, 16), :]
            ibin = 16 * q + iota
            for a, icl, ilane in diag:
                vals = plsc.load_gather(rv, [iota, ilane])
                tv = tile_v.at[b, pl.ds(a, 16), :]
                plsc.store_scatter(tv, [icl, ibin], vals)
        for d in out_descs(j, b):
            d.start()

    def body(i, carry):
        process(2 * i, 0)
        process(2 * i + 1, 1)
        return carry

    lax.fori_loop(0, U_PER_W // 2, body, 0)
    for j, b in ((U_PER_W - 2, 0), (U_PER_W - 1, 1)):
        for d in out_descs(jnp.int32(j), b):
            d.wait()


def kernel(indices, table):
    packed = _retile_sc(table.T)
    table_rm = packed.reshape(VOCAB_PAD, EMBED_DIM)
    idx_t = indices.T.astype(jnp.int32)
    out5 = _embed_gather(idx_t, table_rm)
    return out5.transpose((2, 4, 0, 1, 3)).reshape(BATCH, FIELDS, EMBED_DIM)
